# baseline (speedup 1.0000x reference)
import jax
import jax.numpy as jnp
from jax.experimental import pallas as pl
from jax.experimental.pallas import tpu as pltpu

_B, _P, _D, _H, _W = 16, 32, 256, 64, 64
_HW = _H * _W
_NROWS = 32
_PAD = 256


def _fold_kernel(e_ref, cw_ref, b_ref, w_ref):
    g = jax.lax.dot_general(
        e_ref[:, :], cw_ref[:, :], (((1,), (0,)), ((), ())),
        preferred_element_type=jnp.float32)
    for k in range(9):
        for l in range(3):
            w_ref[k * 3 + l, :] = g[l, k * _D:(k + 1) * _D]
    w_ref[27, :] = b_ref[0, :]
    for r in range(28, _NROWS):
        w_ref[r, :] = jnp.zeros((_D,), jnp.float32)


def _main_kernel(pts_ref, ptl_ref, box_ref, w_ref, c_ref, out_ref,
                 sp_ref, pm_ref):
    b = pl.program_id(0)
    f32 = jnp.float32

    idx_row = c_ref[0:1, :]
    xn = c_ref[1:2, :]
    yn = c_ref[2:3, :]

    @pl.when(b == 0)
    def _():
        sp_ref[:, 0:_PAD] = jnp.zeros((3, _PAD), f32)
        sp_ref[:, _PAD + _HW:] = jnp.zeros((3, _PAD), f32)
        pm_ref[27:28, :] = jnp.ones((1, _HW), f32)
        pm_ref[28:_NROWS, :] = jnp.zeros((_NROWS - 28, _HW), f32)

    px = pts_ref[0, :, 0:1]
    py = pts_ref[0, :, 1:2]
    fidx = jnp.floor(py * f32(_H - 1)) * f32(_W) + jnp.floor(px * f32(_W - 1))
    hf = jnp.where(idx_row == fidx, f32(1.0), f32(0.0))

    labf = jnp.floor(ptl_ref[0, 0:1, 0:_P])
    m0f = jnp.where(labf == f32(0.0), f32(1.0), f32(0.0))
    lhs = jnp.concatenate([m0f, jnp.ones((1, _P), f32)], axis=0)
    cc = jax.lax.dot_general(
        lhs, hf, (((1,), (0,)), ((), ())),
        preferred_element_type=jnp.float32)
    c0 = cc[0:1, :]
    c1 = cc[1:2, :] - c0

    x1 = box_ref[0, 0:1, 0:1]
    y1 = box_ref[0, 0:1, 1:2]
    x2 = box_ref[0, 0:1, 2:3]
    y2 = box_ref[0, 0:1, 3:4]
    mk = (xn >= x1) & (xn <= x2) & (yn >= y1) & (yn <= y2)
    mv = jnp.where(mk, f32(1.0), f32(0.0))

    sp_ref[0:1, pl.ds(_PAD, _HW)] = c0
    sp_ref[1:2, pl.ds(_PAD, _HW)] = c1
    sp_ref[2:3, pl.ds(_PAD, _HW)] = mv

    for ky in range(3):
        for kx in range(3):
            s = (ky - 1) * _W + (kx - 1)
            k = ky * 3 + kx
            v = sp_ref[0:3, pl.ds(_PAD + s, _HW)]
            if kx == 0:
                pm_ref[3 * k:3 * k + 3, :] = v * c_ref[3:4, :]
            elif kx == 2:
                pm_ref[3 * k:3 * k + 3, :] = v * c_ref[4:5, :]
            else:
                pm_ref[3 * k:3 * k + 3, :] = v

    wv = w_ref[:, :]
    chunk = 512
    for j in range(_HW // chunk):
        pj = pm_ref[:, j * chunk:(j + 1) * chunk]
        oj = jax.lax.dot_general(
            wv, pj, (((0,), (0,)), ((), ())),
            preferred_element_type=jnp.float32)
        out_ref[0, :, j * chunk:(j + 1) * chunk] = oj


def kernel(points, boxes, pt_embed_w, box_token, conv_w, conv_b):
    f32 = jnp.float32
    e_p = jnp.concatenate(
        [pt_embed_w, box_token, jnp.zeros((5, _D), f32)], axis=0)
    cw2 = jnp.transpose(conv_w, (1, 2, 3, 0)).reshape(_D, 9 * _D)
    bias2 = conv_b.reshape(1, _D)
    pts_s = jnp.pad(points, ((0, 0), (0, 0), (0, 125)))
    ptl = jnp.pad(points[:, :, 2].reshape(_B, 1, _P),
                  ((0, 0), (0, 7), (0, 128 - _P)))
    boxes_p = jnp.pad(boxes.reshape(_B, 1, 4),
                      ((0, 0), (0, 7), (0, 124)))

    ii = jnp.arange(_HW, dtype=f32).reshape(1, _HW)
    yrow = jnp.floor(ii / _W)
    xcol = ii - yrow * _W
    consts = jnp.concatenate([
        ii,
        xcol / f32(_W - 1),
        yrow / f32(_H - 1),
        (xcol >= 1.0).astype(f32),
        (xcol <= f32(_W - 2)).astype(f32),
        jnp.zeros((3, _HW), f32),
    ], axis=0)

    wf = pl.pallas_call(
        _fold_kernel,
        out_shape=jax.ShapeDtypeStruct((_NROWS, _D), f32),
    )(e_p, cw2, bias2)

    out_flat = pl.pallas_call(
        _main_kernel,
        grid=(_B,),
        in_specs=[
            pl.BlockSpec((1, _P, 128), lambda b: (b, 0, 0)),
            pl.BlockSpec((1, 8, 128), lambda b: (b, 0, 0)),
            pl.BlockSpec((1, 8, 128), lambda b: (b, 0, 0)),
            pl.BlockSpec((_NROWS, _D), lambda b: (0, 0)),
            pl.BlockSpec((8, _HW), lambda b: (0, 0)),
        ],
        out_specs=pl.BlockSpec((1, _D, _HW), lambda b: (b, 0, 0)),
        out_shape=jax.ShapeDtypeStruct((_B, _D, _HW), f32),
        scratch_shapes=[
            pltpu.VMEM((3, _HW + 2 * _PAD), f32),
            pltpu.VMEM((_NROWS, _HW), f32),
        ],
    )(pts_s, ptl, boxes_p, wf, consts)

    return out_flat.reshape(_B, _D, _H, _W)


# speedup vs baseline: 1.0378x; 1.0378x over previous
import jax
import jax.numpy as jnp
from jax.experimental import pallas as pl
from jax.experimental.pallas import tpu as pltpu

_B, _P, _D, _H, _W = 16, 32, 256, 64, 64
_HW = _H * _W
_NROWS = 32
_PAD = 256


def _fold_kernel(e_ref, cw_ref, b_ref, w_ref):
    g = jax.lax.dot_general(
        e_ref[:, :], cw_ref[:, :], (((1,), (0,)), ((), ())),
        preferred_element_type=jnp.float32)
    for k in range(9):
        for l in range(3):
            w_ref[k * 3 + l, :] = g[l, k * _D:(k + 1) * _D]
    w_ref[27, :] = b_ref[0, :]
    for r in range(28, _NROWS):
        w_ref[r, :] = jnp.zeros((_D,), jnp.float32)


def _main_kernel(pts_ref, ptl_ref, box_ref, w_ref, c_ref, out_ref,
                 sp_ref, pm_ref):
    b = pl.program_id(0)
    f32 = jnp.float32

    idx_row = c_ref[0:1, :]
    xn = c_ref[1:2, :]
    yn = c_ref[2:3, :]

    @pl.when(b == 0)
    def _():
        sp_ref[:, 0:_PAD] = jnp.zeros((3, _PAD), f32)
        sp_ref[:, _PAD + _HW:] = jnp.zeros((3, _PAD), f32)
        pm_ref[27:28, :] = jnp.ones((1, _HW), f32)
        pm_ref[28:_NROWS, :] = jnp.zeros((_NROWS - 28, _HW), f32)

    px = pts_ref[0, :, 0:1]
    py = pts_ref[0, :, 1:2]
    fidx = jnp.floor(py * f32(_H - 1)) * f32(_W) + jnp.floor(px * f32(_W - 1))
    hf = jnp.where(idx_row == fidx, f32(1.0), f32(0.0))

    labf = jnp.floor(ptl_ref[0, 0:1, 0:_P])
    m0f = jnp.where(labf == f32(0.0), f32(1.0), f32(0.0))
    lhs = jnp.concatenate([m0f, jnp.ones((1, _P), f32)], axis=0)
    cc = jax.lax.dot_general(
        lhs, hf, (((1,), (0,)), ((), ())),
        preferred_element_type=jnp.float32)
    c0 = cc[0:1, :]
    c1 = cc[1:2, :] - c0

    x1 = box_ref[0, 0:1, 0:1]
    y1 = box_ref[0, 0:1, 1:2]
    x2 = box_ref[0, 0:1, 2:3]
    y2 = box_ref[0, 0:1, 3:4]
    mk = (xn >= x1) & (xn <= x2) & (yn >= y1) & (yn <= y2)
    mv = jnp.where(mk, f32(1.0), f32(0.0))

    sp_ref[0:1, pl.ds(_PAD, _HW)] = c0
    sp_ref[1:2, pl.ds(_PAD, _HW)] = c1
    sp_ref[2:3, pl.ds(_PAD, _HW)] = mv

    for ky in range(3):
        for kx in range(3):
            s = (ky - 1) * _W + (kx - 1)
            k = ky * 3 + kx
            for l in range(3):
                v = sp_ref[l:l + 1, pl.ds(_PAD + s, _HW)]
                if kx == 0:
                    pm_ref[3 * k + l:3 * k + l + 1, :] = v * c_ref[3:4, :]
                elif kx == 2:
                    pm_ref[3 * k + l:3 * k + l + 1, :] = v * c_ref[4:5, :]
                else:
                    pm_ref[3 * k + l:3 * k + l + 1, :] = v

    wv = w_ref[:, :]
    chunk = 512
    for j in range(_HW // chunk):
        pj = pm_ref[:, j * chunk:(j + 1) * chunk]
        oj = jax.lax.dot_general(
            wv, pj, (((0,), (0,)), ((), ())),
            preferred_element_type=jnp.float32)
        out_ref[0, :, j * chunk:(j + 1) * chunk] = oj


def kernel(points, boxes, pt_embed_w, box_token, conv_w, conv_b):
    f32 = jnp.float32
    e_p = jnp.concatenate(
        [pt_embed_w, box_token, jnp.zeros((5, _D), f32)], axis=0)
    cw2 = jnp.transpose(conv_w, (1, 2, 3, 0)).reshape(_D, 9 * _D)
    bias2 = conv_b.reshape(1, _D)
    pts_s = jnp.pad(points, ((0, 0), (0, 0), (0, 125)))
    ptl = jnp.pad(points[:, :, 2].reshape(_B, 1, _P),
                  ((0, 0), (0, 7), (0, 128 - _P)))
    boxes_p = jnp.pad(boxes.reshape(_B, 1, 4),
                      ((0, 0), (0, 7), (0, 124)))

    ii = jnp.arange(_HW, dtype=f32).reshape(1, _HW)
    yrow = jnp.floor(ii / _W)
    xcol = ii - yrow * _W
    consts = jnp.concatenate([
        ii,
        xcol / f32(_W - 1),
        yrow / f32(_H - 1),
        (xcol >= 1.0).astype(f32),
        (xcol <= f32(_W - 2)).astype(f32),
        jnp.zeros((3, _HW), f32),
    ], axis=0)

    wf = pl.pallas_call(
        _fold_kernel,
        out_shape=jax.ShapeDtypeStruct((_NROWS, _D), f32),
    )(e_p, cw2, bias2)

    out_flat = pl.pallas_call(
        _main_kernel,
        grid=(_B,),
        in_specs=[
            pl.BlockSpec((1, _P, 128), lambda b: (b, 0, 0)),
            pl.BlockSpec((1, 8, 128), lambda b: (b, 0, 0)),
            pl.BlockSpec((1, 8, 128), lambda b: (b, 0, 0)),
            pl.BlockSpec((_NROWS, _D), lambda b: (0, 0)),
            pl.BlockSpec((8, _HW), lambda b: (0, 0)),
        ],
        out_specs=pl.BlockSpec((1, _D, _HW), lambda b: (b, 0, 0)),
        out_shape=jax.ShapeDtypeStruct((_B, _D, _HW), f32),
        scratch_shapes=[
            pltpu.VMEM((3, _HW + 2 * _PAD), f32),
            pltpu.VMEM((_NROWS, _HW), f32),
        ],
    )(pts_s, ptl, boxes_p, wf, consts)

    return out_flat.reshape(_B, _D, _H, _W)


# speedup vs baseline: 1.0641x; 1.0253x over previous
import jax
import jax.numpy as jnp
from jax.experimental import pallas as pl
from jax.experimental.pallas import tpu as pltpu

_B, _P, _D, _H, _W = 16, 32, 256, 64, 64
_HW = _H * _W
_NROWS = 32


def _fold_kernel(e_ref, cw_ref, b_ref, w_ref):
    g = jax.lax.dot_general(
        e_ref[:, :], cw_ref[:, :], (((1,), (0,)), ((), ())),
        preferred_element_type=jnp.float32)
    for l in range(3):
        for k in range(9):
            w_ref[l * 9 + k, :] = g[l, k * _D:(k + 1) * _D]
    w_ref[27, :] = b_ref[0, :]
    for r in range(28, _NROWS):
        w_ref[r, :] = jnp.zeros((_D,), jnp.float32)


def _main_kernel(pts_ref, box_ref, w_ref, out_ref, sp_ref, pm_ref):
    f32 = jnp.float32

    @pl.when(pl.program_id(0) == 0)
    def _():
        sp_ref[:, :] = jnp.zeros(sp_ref.shape, f32)
        pm_ref[27:28, :] = jnp.ones((1, _HW), f32)
        pm_ref[28:_NROWS, :] = jnp.zeros((_NROWS - 28, _HW), f32)

    px = pts_ref[0, :, 0:1]
    py = pts_ref[0, :, 1:2]
    pz = pts_ref[0, :, 2:3]
    ix = jnp.floor(px * f32(_W - 1))
    iy = jnp.floor(py * f32(_H - 1))
    lab = jnp.floor(pz)
    fidx = iy * f32(_W) + ix

    ii = jax.lax.broadcasted_iota(jnp.int32, (_P, _HW), 1).astype(f32)
    hit = (ii == fidx)
    m0 = (lab == f32(0.0))
    m1 = (lab == f32(1.0))
    c0 = jnp.sum(jnp.where(hit & m0, f32(1.0), f32(0.0)), axis=0, keepdims=True)
    c1 = jnp.sum(jnp.where(hit & m1, f32(1.0), f32(0.0)), axis=0, keepdims=True)

    li = jax.lax.broadcasted_iota(jnp.int32, (1, _HW), 1).astype(f32)
    yrow = jnp.floor(li * f32(1.0 / _W))
    xcol = li - yrow * f32(_W)
    xn = xcol / f32(_W - 1)
    yn = yrow / f32(_H - 1)
    x1 = box_ref[0, 0:1, 0:1]
    y1 = box_ref[0, 0:1, 1:2]
    x2 = box_ref[0, 0:1, 2:3]
    y2 = box_ref[0, 0:1, 3:4]
    mk = (xn >= x1) & (xn <= x2) & (yn >= y1) & (yn <= y2)
    mv = jnp.where(mk, f32(1.0), f32(0.0))

    sp_ref[0:1, pl.ds(256, _HW)] = c0
    sp_ref[1:2, pl.ds(256, _HW)] = c1
    sp_ref[2:3, pl.ds(256, _HW)] = mv

    xm = {
        -1: jnp.where(xcol >= f32(1.0), f32(1.0), f32(0.0)),
        0: jnp.ones((1, _HW), f32),
        1: jnp.where(xcol <= f32(_W - 2), f32(1.0), f32(0.0)),
    }

    for l in range(3):
        for ky in range(3):
            for kx in range(3):
                s = (ky - 1) * _W + (kx - 1)
                row = l * 9 + ky * 3 + kx
                v = sp_ref[l:l + 1, pl.ds(256 + s, _HW)]
                pm_ref[row:row + 1, :] = v * xm[kx - 1]

    wv = w_ref[:, :]
    chunk = 512
    for j in range(_HW // chunk):
        pj = pm_ref[:, j * chunk:(j + 1) * chunk]
        oj = jax.lax.dot_general(
            wv, pj, (((0,), (0,)), ((), ())),
            preferred_element_type=jnp.float32)
        out_ref[0, :, j * chunk:(j + 1) * chunk] = oj


def kernel(points, boxes, pt_embed_w, box_token, conv_w, conv_b):
    f32 = jnp.float32
    e_p = jnp.concatenate(
        [pt_embed_w, box_token, jnp.zeros((5, _D), f32)], axis=0)
    cw2 = jnp.transpose(conv_w, (1, 2, 3, 0)).reshape(_D, 9 * _D)
    bias2 = conv_b.reshape(1, _D)
    pts_s = jnp.pad(points, ((0, 0), (0, 0), (0, 125)))
    boxes_p = jnp.pad(boxes.reshape(_B, 1, 4),
                      ((0, 0), (0, 7), (0, 124)))

    wf = pl.pallas_call(
        _fold_kernel,
        out_shape=jax.ShapeDtypeStruct((_NROWS, _D), f32),
    )(e_p, cw2, bias2)

    out_flat = pl.pallas_call(
        _main_kernel,
        grid=(_B,),
        in_specs=[
            pl.BlockSpec((1, _P, 128), lambda b: (b, 0, 0)),
            pl.BlockSpec((1, 8, 128), lambda b: (b, 0, 0)),
            pl.BlockSpec((_NROWS, _D), lambda b: (0, 0)),
        ],
        out_specs=pl.BlockSpec((1, _D, _HW), lambda b: (b, 0, 0)),
        out_shape=jax.ShapeDtypeStruct((_B, _D, _HW), f32),
        scratch_shapes=[
            pltpu.VMEM((8, _HW + 512), f32),
            pltpu.VMEM((_NROWS, _HW), f32),
        ],
    )(pts_s, boxes_p, wf)

    return out_flat.reshape(_B, _D, _H, _W)
